# Initial kernel scaffold; baseline (speedup 1.0000x reference)
import jax
import jax.numpy as jnp
from jax.experimental import pallas as pl


def kernel(xyz, sa1_W0, sa1_b0, sa1_g0, sa1_bt0, sa1_W1, sa1_b1, sa1_g1, sa1_bt1, sa1_W2, sa1_b2, sa1_g2, sa1_bt2, sa2_W0, sa2_b0, sa2_g0, sa2_bt0, sa2_W1, sa2_b1, sa2_g1, sa2_bt1, sa2_W2, sa2_b2, sa2_g2, sa2_bt2, sa3_W0, sa3_b0, sa3_g0, sa3_bt0, sa3_W1, sa3_b1, sa3_g1, sa3_bt1, sa3_W2, sa3_b2, sa3_g2, sa3_bt2):
    raise NotImplementedError("write your pallas kernel here")


# speedup vs baseline: 3.2216x; 3.2216x over previous
import functools

import jax
import jax.numpy as jnp
from jax.experimental import pallas as pl

_DEN = 1.0
_HIGH = jax.lax.Precision.HIGHEST


def _fps_body(xyz_ref, new_ref, *, npoint):
    px = xyz_ref[0]
    n = px.shape[1]
    jota = jax.lax.broadcasted_iota(jnp.int32, (1, n), 1).astype(jnp.float32)
    pcols = jax.lax.broadcasted_iota(
        jnp.int32, (1, npoint), 1).astype(jnp.float32)

    def body(i, carry):
        dist, far, newc = carry
        onehot = (jota == far).astype(jnp.float32)
        c = jnp.sum(px * onehot, axis=1, keepdims=True)
        newc = jnp.where(pcols == i.astype(jnp.float32), c, newc)
        d = (px[0:1] - c[0:1]) ** 2
        d = d + (px[1:2] - c[1:2]) ** 2
        d = d + (px[2:3] - c[2:3]) ** 2
        dist = jnp.minimum(dist, d)
        m = jnp.max(dist)
        far = jnp.min(jnp.where(dist == m, jota, jnp.float32(n)))
        return dist, far, newc

    init = (
        jnp.full((1, n), 1e10, jnp.float32),
        jnp.float32(0.0),
        jnp.zeros((3, npoint), jnp.float32),
    )
    _, _, newc = jax.lax.fori_loop(0, npoint, body, init)
    new_ref[0] = newc


def _fps(xyz3n, npoint):
    b, _, n = xyz3n.shape
    return pl.pallas_call(
        functools.partial(_fps_body, npoint=npoint),
        grid=(b,),
        in_specs=[pl.BlockSpec((1, 3, n), lambda i: (i, 0, 0))],
        out_specs=pl.BlockSpec((1, 3, npoint), lambda i: (i, 0, 0)),
        out_shape=jax.ShapeDtypeStruct((b, 3, npoint), jnp.float32),
    )(xyz3n)


def _group_body(nxt_ref, px_ref, feats_ref, *refs, ts, ns, r2, ctot, nlayers):
    wrefs = refs[:-1]
    out_ref = refs[-1]
    nxt = nxt_ref[0]
    px = px_ref[0]
    n = px.shape[1]

    sqd = (nxt[:, 0:1] - px[0:1]) ** 2
    sqd = sqd + (nxt[:, 1:2] - px[1:2]) ** 2
    sqd = sqd + (nxt[:, 2:3] - px[2:3]) ** 2

    inr = sqd <= r2
    maskf = inr.astype(jnp.float32)
    cum = maskf
    sh = 1
    while sh < n:
        cum = cum + jnp.concatenate(
            [jnp.zeros((ts, sh), jnp.float32), cum[:, : n - sh]], axis=1
        )
        sh *= 2
    total = cum[:, n - 1 : n]
    sel = jnp.where(inr, cum - 1.0, jnp.float32(1e9))
    jota = jax.lax.broadcasted_iota(jnp.int32, (1, n), 1).astype(jnp.float32)

    cols = []
    idx0 = None
    for k in range(ns):
        ind = (sel == jnp.float32(k)).astype(jnp.float32)
        sj = jnp.sum(ind * jota, axis=1, keepdims=True)
        if k == 0:
            idx0 = sj
            cols.append(sj)
        else:
            cols.append(jnp.where(total > jnp.float32(k), sj, idx0))
    idx = jnp.concatenate(cols, axis=1)

    oh = (idx[:, :, None] == jota[None, :, :]).astype(jnp.float32)
    oh = oh.reshape(ts * ns, n)
    f = feats_ref[0]
    h = jnp.dot(oh, f, preferred_element_type=jnp.float32, precision=_HIGH)

    ctr = jnp.broadcast_to(nxt[:, None, :], (ts, ns, 3)).reshape(ts * ns, 3)
    if ctot > 3:
        ctr = jnp.concatenate(
            [ctr, jnp.zeros((ts * ns, ctot - 3), jnp.float32)], axis=1
        )
    h = h - ctr

    den = jnp.sqrt(jnp.float32(1.0 + 1e-5))
    for li in range(nlayers):
        wt, bb, gg, bt = wrefs[4 * li : 4 * li + 4]
        h = jnp.dot(h[:], wt[:], preferred_element_type=jnp.float32,
                    precision=_HIGH) + bb[:]
        h = jnp.maximum(gg[:] * h / den + bt[:], 0.0)

    oco = h.shape[1]
    out_ref[0] = jnp.max(h.reshape(ts, ns, oco), axis=1)


def _group_mlp(new_xyz_t, xyz3n, feats, params, *, ts, ns, r2):
    b, s, _ = new_xyz_t.shape
    n = xyz3n.shape[2]
    ctot = feats.shape[2]
    nlayers = len(params)
    oco = params[-1][0].shape[0]

    wargs = []
    wspecs = []
    for (w, bb, gg, bt) in params:
        o = w.shape[0]
        wargs += [w.T, bb.reshape(1, o), gg.reshape(1, o), bt.reshape(1, o)]
        wspecs += [
            pl.BlockSpec(w.T.shape, lambda i, j: (0, 0)),
            pl.BlockSpec((1, o), lambda i, j: (0, 0)),
            pl.BlockSpec((1, o), lambda i, j: (0, 0)),
            pl.BlockSpec((1, o), lambda i, j: (0, 0)),
        ]

    return pl.pallas_call(
        functools.partial(
            _group_body, ts=ts, ns=ns, r2=r2, ctot=ctot, nlayers=nlayers
        ),
        grid=(b, s // ts),
        in_specs=[
            pl.BlockSpec((1, ts, 3), lambda i, j: (i, j, 0)),
            pl.BlockSpec((1, 3, n), lambda i, j: (i, 0, 0)),
            pl.BlockSpec((1, n, ctot), lambda i, j: (i, 0, 0)),
        ] + wspecs,
        out_specs=pl.BlockSpec((1, ts, oco), lambda i, j: (i, j, 0)),
        out_shape=jax.ShapeDtypeStruct((b, s, oco), jnp.float32),
    )(new_xyz_t, xyz3n, feats, *wargs)


def kernel(xyz, sa1_W0, sa1_b0, sa1_g0, sa1_bt0, sa1_W1, sa1_b1, sa1_g1,
           sa1_bt1, sa1_W2, sa1_b2, sa1_g2, sa1_bt2, sa2_W0, sa2_b0, sa2_g0,
           sa2_bt0, sa2_W1, sa2_b1, sa2_g1, sa2_bt1, sa2_W2, sa2_b2, sa2_g2,
           sa2_bt2, sa3_W0, sa3_b0, sa3_g0, sa3_bt0, sa3_W1, sa3_b1, sa3_g1,
           sa3_bt1, sa3_W2, sa3_b2, sa3_g2, sa3_bt2):
    npoint = 128
    p1 = [(sa1_W0, sa1_b0, sa1_g0, sa1_bt0), (sa1_W1, sa1_b1, sa1_g1, sa1_bt1),
          (sa1_W2, sa1_b2, sa1_g2, sa1_bt2)]
    p2 = [(sa2_W0, sa2_b0, sa2_g0, sa2_bt0), (sa2_W1, sa2_b1, sa2_g1, sa2_bt1),
          (sa2_W2, sa2_b2, sa2_g2, sa2_bt2)]
    p3 = [(sa3_W0, sa3_b0, sa3_g0, sa3_bt0), (sa3_W1, sa3_b1, sa3_g1, sa3_bt1),
          (sa3_W2, sa3_b2, sa3_g2, sa3_bt2)]

    xyz1 = xyz
    xyz1_t = jnp.transpose(xyz1, (0, 2, 1))

    nxz1 = _fps(xyz1, npoint)
    nxz1_t = jnp.transpose(nxz1, (0, 2, 1))
    r2_1 = float(0.2 ** 2)
    l1 = _group_mlp(nxz1_t, xyz1, xyz1_t, p1, ts=8, ns=32, r2=r2_1)

    nxz2 = _fps(nxz1, npoint)
    nxz2_t = jnp.transpose(nxz2, (0, 2, 1))
    feats2 = jnp.concatenate([nxz1_t, l1], axis=2)
    r2_2 = float(0.4 ** 2)
    l2 = _group_mlp(nxz2_t, nxz1, feats2, p2, ts=32, ns=64, r2=r2_2)

    nxz3 = _fps(nxz2, npoint)
    nxz3_t = jnp.transpose(nxz3, (0, 2, 1))
    feats3 = jnp.concatenate([nxz2_t, l2], axis=2)
    l3 = _group_mlp(nxz3_t, nxz2, feats3, p3, ts=32, ns=64, r2=r2_2)

    return l3
